# Initial kernel scaffold; baseline (speedup 1.0000x reference)
import jax
import jax.numpy as jnp
from jax.experimental import pallas as pl


def kernel(features, edge_index, W1, b1, W2, b2):
    raise NotImplementedError("write your pallas kernel here")


# speedup vs baseline: 167.1178x; 167.1178x over previous
import functools

import jax
import jax.numpy as jnp
from jax import lax
from jax.experimental import pallas as pl
from jax.experimental.pallas import tpu as pltpu
from jax.experimental.pallas import tpu_sc as plsc

NC = 2
NS = 16
NW = NC * NS
LANES = 16


def _seg_partial_kernel(n_pad, n_edges, chunk):
    e_per_w = n_edges // NW
    n_chunks = e_per_w // chunk
    sl = n_pad // NS

    mesh = plsc.VectorSubcoreMesh(core_axis_name="c", subcore_axis_name="s")

    @functools.partial(
        pl.kernel,
        out_type=jax.ShapeDtypeStruct((NC * n_pad,), jnp.float32),
        mesh=mesh,
        scratch_types=[
            pltpu.VMEM((n_pad,), jnp.float32),
            pltpu.VMEM((chunk,), jnp.int32),
            pltpu.VMEM((chunk,), jnp.int32),
            pltpu.VMEM((chunk,), jnp.float32),
            pltpu.VMEM((n_pad // NS,), jnp.float32),
            pltpu.VMEM_SHARED((n_pad,), jnp.float32),
        ],
        compiler_params=pltpu.CompilerParams(needs_layout_passes=False),
    )
    def seg(vals_hbm, src_hbm, dst_hbm, out_hbm, table_v, sidx_v, didx_v,
            vals_v, z_v, acc_sh):
        cid = lax.axis_index("c")
        sid = lax.axis_index("s")
        wid = cid * NS + sid

        def zero_body(i, carry):
            z_v[pl.ds(i * LANES, LANES)] = jnp.zeros((LANES,), jnp.float32)
            return carry
        lax.fori_loop(0, sl // LANES, zero_body, 0)
        abase = pl.multiple_of(sid * sl, 8)
        pltpu.sync_copy(z_v, acc_sh.at[pl.ds(abase, sl)])

        pltpu.sync_copy(vals_hbm, table_v)

        plsc.subcore_barrier()

        ebase = wid * e_per_w

        def chunk_body(ci, carry):
            off = pl.multiple_of(ebase + ci * chunk, 8)
            pltpu.sync_copy(src_hbm.at[pl.ds(off, chunk)], sidx_v)
            pltpu.sync_copy(dst_hbm.at[pl.ds(off, chunk)], didx_v)

            def gather_body(i, c2):
                idx16 = sidx_v[pl.ds(i * LANES, LANES)]
                vals_v[pl.ds(i * LANES, LANES)] = plsc.load_gather(
                    table_v, [idx16])
                return c2
            lax.fori_loop(0, chunk // LANES, gather_body, 0, unroll=4)

            pltpu.sync_copy(vals_v, acc_sh.at[didx_v], add=True)
            return carry
        lax.fori_loop(0, n_chunks, chunk_body, 0)

        plsc.subcore_barrier()

        obase = pl.multiple_of(cid * n_pad + sid * sl, 8)
        pltpu.sync_copy(acc_sh.at[pl.ds(abase, sl)], z_v)
        pltpu.sync_copy(z_v, out_hbm.at[pl.ds(obase, sl)])

    return seg


def _pw1_body(p_ref, w1_ref, b1_ref, w2_ref, y_ref):
    s = p_ref[0] + p_ref[1]
    acc = jnp.zeros_like(s)
    for k in range(16):
        acc = acc + jnp.maximum(s * w1_ref[0, k] + b1_ref[k], 0.0) * w2_ref[k, 0]
    y_ref[...] = acc


def _pw2_body(q_ref, b2_ref, o_ref):
    o_ref[...] = jnp.maximum(q_ref[0] + q_ref[1] + b2_ref[0], 0.0)


def kernel(features, edge_index, W1, b1, W2, b2):
    n = features.shape[0]
    e = edge_index.shape[1]
    assert e % NW == 0
    e_per_w = e // NW

    chunk = None
    for c in (4000, 2000, 1000, 16):
        if e_per_w % c == 0 and c % LANES == 0:
            chunk = c
            break
    assert chunk is not None and e_per_w % chunk == 0

    n_pad = ((n + 127) // 128) * 128
    rows = n_pad // 128

    feat = jnp.pad(features[:, 0], (0, n_pad - n))
    src = edge_index[0].astype(jnp.int32)
    dst = edge_index[1].astype(jnp.int32)

    seg = _seg_partial_kernel(n_pad, e, chunk)

    pw1 = pl.pallas_call(
        _pw1_body,
        out_shape=jax.ShapeDtypeStruct((rows, 128), jnp.float32),
        in_specs=[
            pl.BlockSpec(memory_space=pltpu.VMEM),
            pl.BlockSpec(memory_space=pltpu.SMEM),
            pl.BlockSpec(memory_space=pltpu.SMEM),
            pl.BlockSpec(memory_space=pltpu.SMEM),
        ],
        out_specs=pl.BlockSpec(memory_space=pltpu.VMEM),
    )
    pw2 = pl.pallas_call(
        _pw2_body,
        out_shape=jax.ShapeDtypeStruct((rows, 128), jnp.float32),
        in_specs=[
            pl.BlockSpec(memory_space=pltpu.VMEM),
            pl.BlockSpec(memory_space=pltpu.SMEM),
        ],
        out_specs=pl.BlockSpec(memory_space=pltpu.VMEM),
    )

    p = seg(feat, src, dst)
    y = pw1(p.reshape(NC, rows, 128), W1, b1, W2)
    q = seg(y.reshape(n_pad), src, dst)
    out = pw2(q.reshape(NC, rows, 128), b2)
    return out.reshape(n_pad)[:n, None]
